# baseline (speedup 1.0000x reference)
import functools
import math

import numpy as np
import jax
import jax.numpy as jnp
from jax import lax
from jax.experimental import pallas as pl
from jax.experimental.pallas import tpu as pltpu
from jax.experimental.pallas import tpu_sc as plsc

_NUM_LEVELS = 16
_LEVEL_DIM = 2
_BASE_RES = 16
_LOG2_HASHMAP = 19
_DESIRED_RES = 2048
_BOUND = 1.6
_N_POINTS = 524288
_PER_LEVEL_SCALE = 2.0 ** (np.log2(_DESIRED_RES / float(_BASE_RES)) / (_NUM_LEVELS - 1))
_MAX_PARAMS = 2 ** _LOG2_HASHMAP

def _i32(v):
    return ((v + 2 ** 31) % 2 ** 32) - 2 ** 31


_H2 = _i32(2654435761)
_H3 = 805459861
_H2D = _i32(2 * 2654435761)
_H3D = _i32(2 * 805459861)
_MASK2 = (_MAX_PARAMS - 1) << 1


def _level_table():
    levels = []
    offset = 0
    for l in range(_NUM_LEVELS):
        scale = _BASE_RES * (_PER_LEVEL_SCALE ** l) - 1.0
        res = int(math.ceil(scale)) + 1
        params = min(_MAX_PARAMS, res ** 3)
        params = int((params + 7) // 8 * 8)
        levels.append((float(scale), res, params, offset, res ** 3 <= params))
        offset += params
    return levels, offset


_LEVELS, _TOTAL_PARAMS = _level_table()
_N_PERSIST = 3
_DENSE_WORDS = 2 * _LEVELS[_N_PERSIST][3]
_HASH_WORDS = 2 * _MAX_PARAMS

_NW = 32
_NS = 16
_C = 1024
_PPW = _N_POINTS // _NW
_NSTEP = _PPW // _C
_F = _NUM_LEVELS * _LEVEL_DIM

def _stage_chunk(words):
    return ((words + 8 * _NS - 1) // (8 * _NS)) * 8


_DENSE_CH = _stage_chunk(_DENSE_WORDS)
_DENSE_PAD = _DENSE_CH * _NS


def _sc_body(xyzf, emb, out, xyzt, xb, yb, zb, idxb, wb, gb, ob, didx,
             shd, shh, shb):
    sid = lax.axis_index("s")
    cid = lax.axis_index("c")
    wid = sid * 2 + cid
    base0 = wid * _PPW
    iot = lax.iota(jnp.int32, 16)

    def stage(dst, src_off, dst_off, words):
        done = 0
        while done < words:
            n = min(16 * _C, words - done)
            pltpu.sync_copy(emb.at[pl.ds(src_off + done, n)], gb.at[pl.ds(0, n)])
            pltpu.sync_copy(gb.at[pl.ds(0, n)], dst.at[pl.ds(dst_off + done, n)])
            done += n

    def didx_body(j, c):
        o = j * 16
        v = o + iot
        didx[pl.ds(o, 16)] = sid * (3 * _C) + 3 * (v & (_C - 1)) + (v >> 10)
        return c

    lax.fori_loop(0, 3 * _C // 16, didx_body, 0)

    def deint_body(s, c):
        base = base0 + s * _C
        pltpu.sync_copy(xyzf.at[pl.ds(3 * base, 3 * _C)], wb.at[pl.ds(0, 3 * _C)])
        pltpu.sync_copy(wb.at[pl.ds(0, 3 * _C)], shb.at[pl.ds(sid * 3 * _C, 3 * _C)])
        pltpu.sync_copy(shb.at[didx], gb.at[pl.ds(0, 3 * _C)])
        for d in range(3):
            pltpu.sync_copy(gb.at[pl.ds(d * _C, _C)],
                            xyzt.at[pl.ds(d * _N_POINTS + base, _C)])
        return c

    lax.fori_loop(0, _NSTEP, deint_body, 0)

    stage(shd, sid * _DENSE_CH, sid * _DENSE_CH, _DENSE_CH)
    plsc.subcore_barrier()

    def run_level(lvl, scale, res, offset, dense):
        persist = lvl < _N_PERSIST
        src = shd if persist else shh
        db = 2 * offset if persist else 0
        sc = float(np.float32(scale) / np.float32(2.0 * _BOUND))
        bi = float(np.float32(_BOUND) * np.float32(sc) + np.float32(0.5))

        def chunk_fn(s, carry):
            cb = s * _C
            base = base0 + cb
            pltpu.sync_copy(xyzt.at[pl.ds(base, _C)], xb)
            pltpu.sync_copy(xyzt.at[pl.ds(_N_POINTS + base, _C)], yb)
            pltpu.sync_copy(xyzt.at[pl.ds(2 * _N_POINTS + base, _C)], zb)

            def comp_body(j, c):
                o = j * 16
                px = xb[pl.ds(o, 16)] * sc + bi
                py = yb[pl.ds(o, 16)] * sc + bi
                pz = zb[pl.ds(o, 16)] * sc + bi
                gx = px.astype(jnp.int32)
                gy = py.astype(jnp.int32)
                gz = pz.astype(jnp.int32)
                fx1 = px - gx.astype(jnp.float32)
                fy1 = py - gy.astype(jnp.float32)
                fz1 = pz - gz.astype(jnp.float32)
                fx0 = 1.0 - fx1
                fy0 = 1.0 - fy1
                fz0 = 1.0 - fz1
                if dense:
                    x0 = gx + gx
                    tx = (x0, x0 + 2)
                    y0 = gy * (2 * res)
                    ty = (y0, y0 + 2 * res)
                    z0 = gz * (2 * res * res) + db
                    tz = (z0, z0 + 2 * res * res)
                else:
                    x0 = gx + gx
                    tx = (x0, x0 + 2)
                    y0 = gy * _H2D
                    ty = (y0, y0 + _H2D)
                    z0 = gz * _H3D
                    tz = (z0, z0 + _H3D)
                fx = (fx0, fx1)
                fy = (fy0, fy1)
                fz = (fz0, fz1)
                oj = j * 16
                for corner in range(8):
                    bx, by, bz = corner & 1, (corner >> 1) & 1, (corner >> 2) & 1
                    w = fx[bx] * fy[by]
                    w = w * fz[bz]
                    if dense:
                        i2 = tx[bx] + ty[by] + tz[bz]
                    else:
                        i2 = (tx[bx] ^ ty[by] ^ tz[bz]) & _MASK2
                    idxb[pl.ds(corner * _C + oj, 16)] = i2
                    idxb[pl.ds(8 * _C + corner * _C + oj, 16)] = i2 + 1
                    wb[pl.ds(corner * _C + oj, 16)] = w
                return c

            lax.fori_loop(0, _C // 16, comp_body, 0)

            pltpu.sync_copy(src.at[idxb], gb)

            def acc_body(j, c):
                o = j * 16
                acc0 = jnp.full((16,), 0.0, jnp.float32)
                acc1 = jnp.full((16,), 0.0, jnp.float32)
                for corner in range(8):
                    g0 = gb[pl.ds(corner * _C + o, 16)]
                    g1 = gb[pl.ds(8 * _C + corner * _C + o, 16)]
                    wv = wb[pl.ds(corner * _C + o, 16)]
                    acc0 = acc0 + g0 * wv
                    acc1 = acc1 + g1 * wv
                ob[pl.ds(o, 16)] = acc0
                ob[pl.ds(_C + o, 16)] = acc1
                return c

            lax.fori_loop(0, _C // 16, acc_body, 0)

            pltpu.sync_copy(
                ob.at[pl.ds(0, _C)],
                out.at[pl.ds((2 * lvl) * _N_POINTS + base, _C)],
            )
            pltpu.sync_copy(
                ob.at[pl.ds(_C, _C)],
                out.at[pl.ds((2 * lvl + 1) * _N_POINTS + base, _C)],
            )
            return carry

        lax.fori_loop(0, _NSTEP, chunk_fn, 0)

    for lvl, (scale, res, params, offset, dense) in enumerate(_LEVELS):
        if lvl >= _N_PERSIST:
            ch = _stage_chunk(2 * params)
            plsc.subcore_barrier()
            stage(shh, 2 * offset + sid * ch, sid * ch, ch)
            plsc.subcore_barrier()
        run_level(lvl, scale, res, offset, dense)


@functools.partial(
    pl.kernel,
    out_type=(
        jax.ShapeDtypeStruct((_N_POINTS * _F,), jnp.float32),
        jax.ShapeDtypeStruct((3 * _N_POINTS,), jnp.float32),
    ),
    mesh=plsc.VectorSubcoreMesh(
        core_axis_name="c", subcore_axis_name="s", num_cores=2, num_subcores=16
    ),
    scratch_types=[
        pltpu.VMEM((_C,), jnp.float32),
        pltpu.VMEM((_C,), jnp.float32),
        pltpu.VMEM((_C,), jnp.float32),
        pltpu.VMEM((16 * _C,), jnp.int32),
        pltpu.VMEM((8 * _C,), jnp.float32),
        pltpu.VMEM((16 * _C,), jnp.float32),
        pltpu.VMEM((2 * _C,), jnp.float32),
        pltpu.VMEM((3 * _C,), jnp.int32),
        pltpu.VMEM_SHARED((_DENSE_PAD,), jnp.float32),
        pltpu.VMEM_SHARED((_HASH_WORDS,), jnp.float32),
        pltpu.VMEM_SHARED((_NS * 3 * _C,), jnp.float32),
    ],
)
def _encode(xyzf, emb, out, xyzt, xb, yb, zb, idxb, wb, gb, ob, didx,
            shd, shh, shb):
    _sc_body(xyzf, emb, out, xyzt, xb, yb, zb, idxb, wb, gb, ob, didx,
             shd, shh, shb)


_TB = 8192


def _tr_kernel(inb, outb):
    outb[...] = inb[...].T


_transpose = pl.pallas_call(
    _tr_kernel,
    grid=(_N_POINTS // _TB,),
    in_specs=[pl.BlockSpec((_F, _TB), lambda i: (0, i))],
    out_specs=pl.BlockSpec((_TB, _F), lambda i: (i, 0)),
    out_shape=jax.ShapeDtypeStruct((_N_POINTS, _F), jnp.float32),
)


def kernel(xyz, embeddings):
    flat, _ = _encode(xyz.reshape(-1), embeddings.reshape(-1))
    return _transpose(flat.reshape(_F, _N_POINTS))


# speedup vs baseline: 4.1966x; 4.1966x over previous
import functools
import math

import numpy as np
import jax
import jax.numpy as jnp
from jax import lax
from jax.experimental import pallas as pl
from jax.experimental.pallas import tpu as pltpu
from jax.experimental.pallas import tpu_sc as plsc

_NUM_LEVELS = 16
_LEVEL_DIM = 2
_BASE_RES = 16
_LOG2_HASHMAP = 19
_DESIRED_RES = 2048
_BOUND = 1.6
_N_POINTS = 524288
_PER_LEVEL_SCALE = 2.0 ** (np.log2(_DESIRED_RES / float(_BASE_RES)) / (_NUM_LEVELS - 1))
_MAX_PARAMS = 2 ** _LOG2_HASHMAP


def _i32(v):
    return ((v + 2 ** 31) % 2 ** 32) - 2 ** 31


_H2 = _i32(2654435761)
_H3 = 805459861
_MASK = _MAX_PARAMS - 1


def _level_table():
    levels = []
    offset = 0
    for l in range(_NUM_LEVELS):
        scale = _BASE_RES * (_PER_LEVEL_SCALE ** l) - 1.0
        res = int(math.ceil(scale)) + 1
        params = min(_MAX_PARAMS, res ** 3)
        params = int((params + 7) // 8 * 8)
        levels.append((float(scale), res, params, offset, res ** 3 <= params))
        offset += params
    return levels, offset


_LEVELS, _TOTAL_PARAMS = _level_table()
_N_PERSIST = 3

_NW = 32
_NS = 16
_C = 1024
_PPW = _N_POINTS // _NW
_NSTEP = _PPW // _C
_F = _NUM_LEVELS * _LEVEL_DIM

_EMB_BLOCKS = (_TOTAL_PARAMS + 127) // 128
_EMB_BLOCKS_PAD = _EMB_BLOCKS + 2
_EMB_WORDS = _EMB_BLOCKS_PAD * 256

_N_BLOCKS = _N_POINTS // 128


def _stage_chunk(words):
    return ((words + 8 * _NS - 1) // (8 * _NS)) * 8


def _staged_words(offset, params):
    b0 = offset // 128
    b1 = -((offset + params) // -128)
    return b0, 256 * (b1 - b0)


_DENSE_WORDS = _staged_words(0, _LEVELS[_N_PERSIST][3])[1]
_DENSE_CH = _stage_chunk(_DENSE_WORDS)
_SLOT_WORDS = max(
    _stage_chunk(_staged_words(_LEVELS[l][3], _LEVELS[l][2])[1]) * _NS
    for l in range(_N_PERSIST, _NUM_LEVELS)
)


def _sc_body(xyzf, emb, out, xyzb, idxb, wb, gb, ob0, ob1, shd, shh):
    sid = lax.axis_index("s")
    cid = lax.axis_index("c")
    wid = sid * 2 + cid
    base0 = wid * _PPW

    def stage(dst, src_off, dst_off, words):
        done = 0
        while done < words:
            n = min(16 * _C, words - done)
            pltpu.sync_copy(emb.at[pl.ds(src_off + done, n)], gb.at[pl.ds(0, n)])
            pltpu.sync_copy(gb.at[pl.ds(0, n)], dst.at[pl.ds(dst_off + done, n)])
            done += n

    stage(shd, sid * _DENSE_CH, sid * _DENSE_CH, _DENSE_CH)
    plsc.subcore_barrier()

    def run_level(lvl, scale, res, offset, dense):
        persist = lvl < _N_PERSIST
        src = shd if persist else shh
        b0 = 0 if persist else _staged_words(offset, _LEVELS[lvl][2])[0]
        sub = 256 * b0
        sc = float(np.float32(scale) / np.float32(2.0 * _BOUND))
        bi = float(np.float32(_BOUND) * np.float32(sc) + np.float32(0.5))

        def chunk_fn(s, carry):
            cb = s * _C
            base = base0 + cb
            pltpu.sync_copy(xyzf.at[pl.ds(4 * base, 4 * _C)], xyzb)

            def comp_body(j, c):
                o = j * 16
                ox = ((o >> 7) << 9) + (o & 127)
                px = xyzb[pl.ds(ox, 16)] * sc + bi
                py = xyzb[pl.ds(ox + 128, 16)] * sc + bi
                pz = xyzb[pl.ds(ox + 256, 16)] * sc + bi
                gx = px.astype(jnp.int32)
                gy = py.astype(jnp.int32)
                gz = pz.astype(jnp.int32)
                fx1 = px - gx.astype(jnp.float32)
                fy1 = py - gy.astype(jnp.float32)
                fz1 = pz - gz.astype(jnp.float32)
                fx0 = 1.0 - fx1
                fy0 = 1.0 - fy1
                fz0 = 1.0 - fz1
                if dense:
                    tx = (gx, gx + 1)
                    y0 = gy * res
                    ty = (y0, y0 + res)
                    z0 = gz * (res * res) + offset
                    tz = (z0, z0 + res * res)
                else:
                    tx = (gx, gx + 1)
                    y0 = gy * _H2
                    ty = (y0, y0 + _H2)
                    z0 = gz * _H3
                    tz = (z0, z0 + _H3)
                fx = (fx0, fx1)
                fy = (fy0, fy1)
                fz = (fz0, fz1)
                for corner in range(8):
                    bx, by, bz = corner & 1, (corner >> 1) & 1, (corner >> 2) & 1
                    w = fx[bx] * fy[by]
                    w = w * fz[bz]
                    if dense:
                        p = tx[bx] + ty[by] + tz[bz]
                    else:
                        p = ((tx[bx] ^ ty[by] ^ tz[bz]) & _MASK) + offset
                    addr = p + (p & (-128))
                    if sub:
                        addr = addr - sub
                    idxb[pl.ds(corner * _C + o, 16)] = addr
                    idxb[pl.ds(8 * _C + corner * _C + o, 16)] = addr + 128
                    wb[pl.ds(corner * _C + o, 16)] = w
                return c

            lax.fori_loop(0, _C // 16, comp_body, 0)

            pltpu.sync_copy(src.at[idxb], gb)

            def acc_body(j, c):
                o = j * 16
                acc0 = jnp.full((16,), 0.0, jnp.float32)
                acc1 = jnp.full((16,), 0.0, jnp.float32)
                for corner in range(8):
                    g0 = gb[pl.ds(corner * _C + o, 16)]
                    g1 = gb[pl.ds(8 * _C + corner * _C + o, 16)]
                    wv = wb[pl.ds(corner * _C + o, 16)]
                    acc0 = acc0 + g0 * wv
                    acc1 = acc1 + g1 * wv
                b_ = j >> 3
                l_ = (j & 7) * 16
                ob0[b_, 0, pl.ds(l_, 16)] = acc0
                ob1[b_, 0, pl.ds(l_, 16)] = acc1
                return c

            lax.fori_loop(0, _C // 16, acc_body, 0)

            r0 = base >> 7
            f0, f1 = 2 * lvl, 2 * lvl + 1
            pltpu.sync_copy(
                ob0, out.at[f0 >> 3, pl.ds(r0, 8), pl.ds(f0 & 7, 1)]
            )
            pltpu.sync_copy(
                ob1, out.at[f1 >> 3, pl.ds(r0, 8), pl.ds(f1 & 7, 1)]
            )
            return carry

        lax.fori_loop(0, _NSTEP, chunk_fn, 0)

    for lvl, (scale, res, params, offset, dense) in enumerate(_LEVELS):
        if lvl >= _N_PERSIST:
            b0, words = _staged_words(offset, params)
            ch = _stage_chunk(words)
            plsc.subcore_barrier()
            stage(shh, 256 * b0 + sid * ch, sid * ch, ch)
            plsc.subcore_barrier()
        run_level(lvl, scale, res, offset, dense)


@functools.partial(
    pl.kernel,
    out_type=jax.ShapeDtypeStruct((_F // 8, _N_BLOCKS, 8, 128), jnp.float32),
    mesh=plsc.VectorSubcoreMesh(
        core_axis_name="c", subcore_axis_name="s", num_cores=2, num_subcores=16
    ),
    scratch_types=[
        pltpu.VMEM((4 * _C,), jnp.float32),
        pltpu.VMEM((16 * _C,), jnp.int32),
        pltpu.VMEM((8 * _C,), jnp.float32),
        pltpu.VMEM((16 * _C,), jnp.float32),
        pltpu.VMEM((8, 1, 128), jnp.float32),
        pltpu.VMEM((8, 1, 128), jnp.float32),
        pltpu.VMEM_SHARED((_DENSE_CH * _NS,), jnp.float32),
        pltpu.VMEM_SHARED((_SLOT_WORDS,), jnp.float32),
    ],
)
def _encode(xyzf, emb, out, xyzb, idxb, wb, gb, ob0, ob1, shd, shh):
    _sc_body(xyzf, emb, out, xyzb, idxb, wb, gb, ob0, ob1, shd, shh)


def kernel(xyz, embeddings):
    xyzf = (
        jnp.pad(xyz, ((0, 0), (0, 1)))
        .reshape(_N_BLOCKS, 128, 4)
        .transpose(0, 2, 1)
        .reshape(-1)
    )
    pad_rows = _EMB_BLOCKS_PAD * 128 - _TOTAL_PARAMS
    embf = (
        jnp.pad(embeddings, ((0, pad_rows), (0, 0)))
        .reshape(_EMB_BLOCKS_PAD, 128, 2)
        .transpose(0, 2, 1)
        .reshape(-1)
    )
    out4 = _encode(xyzf, embf)
    return out4.transpose(1, 3, 0, 2).reshape(_N_POINTS, _F)


# speedup vs baseline: 4.2156x; 1.0045x over previous
import functools
import math

import numpy as np
import jax
import jax.numpy as jnp
from jax import lax
from jax.experimental import pallas as pl
from jax.experimental.pallas import tpu as pltpu
from jax.experimental.pallas import tpu_sc as plsc

_NUM_LEVELS = 16
_LEVEL_DIM = 2
_BASE_RES = 16
_LOG2_HASHMAP = 19
_DESIRED_RES = 2048
_BOUND = 1.6
_N_POINTS = 524288
_PER_LEVEL_SCALE = 2.0 ** (np.log2(_DESIRED_RES / float(_BASE_RES)) / (_NUM_LEVELS - 1))
_MAX_PARAMS = 2 ** _LOG2_HASHMAP


def _i32(v):
    return ((v + 2 ** 31) % 2 ** 32) - 2 ** 31


_H2 = _i32(2654435761)
_H3 = 805459861
_MASK = _MAX_PARAMS - 1


def _level_table():
    levels = []
    offset = 0
    for l in range(_NUM_LEVELS):
        scale = _BASE_RES * (_PER_LEVEL_SCALE ** l) - 1.0
        res = int(math.ceil(scale)) + 1
        params = min(_MAX_PARAMS, res ** 3)
        params = int((params + 7) // 8 * 8)
        levels.append((float(scale), res, params, offset, res ** 3 <= params))
        offset += params
    return levels, offset


_LEVELS, _TOTAL_PARAMS = _level_table()
_N_PERSIST = 3

_NW = 32
_NS = 16
_C = 1024
_PPW = _N_POINTS // _NW
_NSTEP = _PPW // _C
_F = _NUM_LEVELS * _LEVEL_DIM

_EMB_BLOCKS = (_TOTAL_PARAMS + 127) // 128
_EMB_BLOCKS_PAD = _EMB_BLOCKS + 2
_EMB_WORDS = _EMB_BLOCKS_PAD * 256

_N_BLOCKS = _N_POINTS // 128


def _stage_chunk(words):
    return ((words + 8 * _NS - 1) // (8 * _NS)) * 8


def _staged_words(offset, params):
    b0 = offset // 128
    b1 = -((offset + params) // -128)
    return b0, 256 * (b1 - b0)


_DENSE_WORDS = _staged_words(0, _LEVELS[_N_PERSIST][3])[1]
_DENSE_CH = _stage_chunk(_DENSE_WORDS)
_SLOT_WORDS = max(
    _stage_chunk(_staged_words(_LEVELS[l][3], _LEVELS[l][2])[1]) * _NS
    for l in range(_N_PERSIST, _NUM_LEVELS)
)


def _sc_body(xyzf, emb, out, xyzb, idxa, idxc, wb, ga, gc, ob0, ob1, sm0, sm1,
             shd, shh):
    sid = lax.axis_index("s")
    cid = lax.axis_index("c")
    wid = sid * 2 + cid
    base0 = wid * _PPW

    def stage(dst, src_off, dst_off, words):
        done = 0
        while done < words:
            n = min(8 * _C, words - done)
            pltpu.sync_copy(emb.at[pl.ds(src_off + done, n)], ga.at[pl.ds(0, n)])
            pltpu.sync_copy(ga.at[pl.ds(0, n)], dst.at[pl.ds(dst_off + done, n)])
            done += n

    stage(shd, sid * _DENSE_CH, sid * _DENSE_CH, _DENSE_CH)
    plsc.subcore_barrier()

    def run_level(lvl, scale, res, offset, dense):
        persist = lvl < _N_PERSIST
        src = shd if persist else shh
        b0 = 0 if persist else _staged_words(offset, _LEVELS[lvl][2])[0]
        sub = 256 * b0
        sc = float(np.float32(scale) / np.float32(2.0 * _BOUND))
        bi = float(np.float32(_BOUND) * np.float32(sc) + np.float32(0.5))

        def chunk_fn(s, carry):
            cb = s * _C
            base = base0 + cb
            pltpu.sync_copy(xyzf.at[pl.ds(4 * base, 4 * _C)], xyzb)

            def comp_body(j, c):
                o = j * 16
                ox = ((o >> 7) << 9) + (o & 127)
                px = xyzb[pl.ds(ox, 16)] * sc + bi
                py = xyzb[pl.ds(ox + 128, 16)] * sc + bi
                pz = xyzb[pl.ds(ox + 256, 16)] * sc + bi
                gx = px.astype(jnp.int32)
                gy = py.astype(jnp.int32)
                gz = pz.astype(jnp.int32)
                fx1 = px - gx.astype(jnp.float32)
                fy1 = py - gy.astype(jnp.float32)
                fz1 = pz - gz.astype(jnp.float32)
                fx0 = 1.0 - fx1
                fy0 = 1.0 - fy1
                fz0 = 1.0 - fz1
                if dense:
                    tx = (gx, gx + 1)
                    y0 = gy * res
                    ty = (y0, y0 + res)
                    z0 = gz * (res * res) + offset
                    tz = (z0, z0 + res * res)
                else:
                    tx = (gx, gx + 1)
                    y0 = gy * _H2
                    ty = (y0, y0 + _H2)
                    z0 = gz * _H3
                    tz = (z0, z0 + _H3)
                fx = (fx0, fx1)
                fy = (fy0, fy1)
                fz = (fz0, fz1)
                for corner in range(8):
                    bx, by, bz = corner & 1, (corner >> 1) & 1, (corner >> 2) & 1
                    w = fx[bx] * fy[by]
                    w = w * fz[bz]
                    if dense:
                        p = tx[bx] + ty[by] + tz[bz]
                    else:
                        p = ((tx[bx] ^ ty[by] ^ tz[bz]) & _MASK) + offset
                    addr = p + (p & (-128))
                    if sub:
                        addr = addr - sub
                    idxa[pl.ds(corner * _C + o, 16)] = addr
                    idxc[pl.ds(corner * _C + o, 16)] = addr + 128
                    wb[pl.ds(corner * _C + o, 16)] = w
                return c

            lax.fori_loop(0, _C // 16, comp_body, 0)

            h0 = pltpu.make_async_copy(src.at[idxa], ga, sm0)
            h0.start()
            h1 = pltpu.make_async_copy(src.at[idxc], gc, sm1)
            h1.start()
            h0.wait()
            h1.wait()

            def acc_body(j, c):
                o = j * 16
                acc0 = jnp.full((16,), 0.0, jnp.float32)
                acc1 = jnp.full((16,), 0.0, jnp.float32)
                for corner in range(8):
                    g0 = ga[pl.ds(corner * _C + o, 16)]
                    g1 = gc[pl.ds(corner * _C + o, 16)]
                    wv = wb[pl.ds(corner * _C + o, 16)]
                    acc0 = acc0 + g0 * wv
                    acc1 = acc1 + g1 * wv
                b_ = j >> 3
                l_ = (j & 7) * 16
                ob0[b_, 0, pl.ds(l_, 16)] = acc0
                ob1[b_, 0, pl.ds(l_, 16)] = acc1
                return c

            lax.fori_loop(0, _C // 16, acc_body, 0)

            r0 = base >> 7
            f0, f1 = 2 * lvl, 2 * lvl + 1
            pltpu.sync_copy(
                ob0, out.at[f0 >> 3, pl.ds(r0, 8), pl.ds(f0 & 7, 1)]
            )
            pltpu.sync_copy(
                ob1, out.at[f1 >> 3, pl.ds(r0, 8), pl.ds(f1 & 7, 1)]
            )
            return carry

        lax.fori_loop(0, _NSTEP, chunk_fn, 0)

    for lvl, (scale, res, params, offset, dense) in enumerate(_LEVELS):
        if lvl >= _N_PERSIST:
            b0, words = _staged_words(offset, params)
            ch = _stage_chunk(words)
            plsc.subcore_barrier()
            stage(shh, 256 * b0 + sid * ch, sid * ch, ch)
            plsc.subcore_barrier()
        run_level(lvl, scale, res, offset, dense)


@functools.partial(
    pl.kernel,
    out_type=jax.ShapeDtypeStruct((_F // 8, _N_BLOCKS, 8, 128), jnp.float32),
    mesh=plsc.VectorSubcoreMesh(
        core_axis_name="c", subcore_axis_name="s", num_cores=2, num_subcores=16
    ),
    scratch_types=[
        pltpu.VMEM((4 * _C,), jnp.float32),
        pltpu.VMEM((8 * _C,), jnp.int32),
        pltpu.VMEM((8 * _C,), jnp.int32),
        pltpu.VMEM((8 * _C,), jnp.float32),
        pltpu.VMEM((8 * _C,), jnp.float32),
        pltpu.VMEM((8 * _C,), jnp.float32),
        pltpu.VMEM((8, 1, 128), jnp.float32),
        pltpu.VMEM((8, 1, 128), jnp.float32),
        pltpu.SemaphoreType.DMA,
        pltpu.SemaphoreType.DMA,
        pltpu.VMEM_SHARED((_DENSE_CH * _NS,), jnp.float32),
        pltpu.VMEM_SHARED((_SLOT_WORDS,), jnp.float32),
    ],
)
def _encode(xyzf, emb, out, xyzb, idxa, idxc, wb, ga, gc, ob0, ob1, sm0, sm1,
            shd, shh):
    _sc_body(xyzf, emb, out, xyzb, idxa, idxc, wb, ga, gc, ob0, ob1, sm0, sm1,
             shd, shh)


def kernel(xyz, embeddings):
    xyzf = (
        jnp.pad(xyz, ((0, 0), (0, 1)))
        .reshape(_N_BLOCKS, 128, 4)
        .transpose(0, 2, 1)
        .reshape(-1)
    )
    pad_rows = _EMB_BLOCKS_PAD * 128 - _TOTAL_PARAMS
    embf = (
        jnp.pad(embeddings, ((0, pad_rows), (0, 0)))
        .reshape(_EMB_BLOCKS_PAD, 128, 2)
        .transpose(0, 2, 1)
        .reshape(-1)
    )
    out4 = _encode(xyzf, embf)
    return out4.transpose(1, 3, 0, 2).reshape(_N_POINTS, _F)
